# baseline (speedup 1.0000x reference)
import functools

import jax
import jax.numpy as jnp
from jax import lax
from jax.experimental import pallas as pl
from jax.experimental.pallas import tpu as pltpu
from jax.experimental.pallas import tpu_sc as plsc

N = 10000
NP = 10240
E = 160000
D = 256
G = 128
HH = 128
BLK = 512
NBLK = NP // BLK

TILES = 16
CK = 80
EPT = E // TILES
NCH = EPT // CK
NHALF = NP // 2
ACCR = 6144
TRASH = 6000
ZPT = ACCR // TILES
WPT = NHALF // TILES
ZCH = 64

_MESH = plsc.VectorSubcoreMesh(core_axis_name="c", subcore_axis_name="s")


def _zero_wide(ref, rows, cols):
    def body(i, _):
        for j in range(cols // 16):
            ref[i, pl.ds(j * 16, 16)] = jnp.zeros((16,), jnp.float32)
        return 0
    lax.fori_loop(0, rows, body, 0)


@functools.partial(
    pl.kernel,
    mesh=_MESH,
    out_type=(
        jax.ShapeDtypeStruct((NP, HH), jnp.float32),
        jax.ShapeDtypeStruct((NP, HH), jnp.float32),
    ),
    scratch_types=[
        pltpu.VMEM((NCH, CK), jnp.int32),
        pltpu.VMEM((NCH, CK), jnp.int32),
        pltpu.VMEM((CK, HH), jnp.float32),
        pltpu.VMEM((CK, HH), jnp.float32),
        pltpu.VMEM((ZCH, HH), jnp.float32),
        pltpu.VMEM_SHARED((ACCR, HH), jnp.float32),
        pltpu.SemaphoreType.DMA,
        pltpu.SemaphoreType.DMA,
    ],
)
def _prop_sc(h0, h1, srcw, dst0w, dst1w, s0, s1, sidx, didx, bufa, bufb, wb,
             acc, sem_a, sem_b):
    cid = lax.axis_index("c")
    sid = lax.axis_index("s")

    pltpu.sync_copy(srcw.at[sid], sidx)

    def do_edges(h):
        pltpu.async_copy(h.at[sidx.at[0]], bufa, sem_a)

        def body(c, _):
            ga = 2 * c
            gb = 2 * c + 1
            pltpu.make_async_copy(h.at[sidx.at[ga]], bufa, sem_a).wait()
            pltpu.async_copy(h.at[sidx.at[gb]], bufb, sem_b)
            pltpu.sync_copy(bufa, acc.at[didx.at[ga]], add=True)
            pltpu.async_copy(h.at[sidx.at[ga + 2]], bufa, sem_a)
            pltpu.make_async_copy(h.at[sidx.at[gb]], bufb, sem_b).wait()
            pltpu.sync_copy(bufb, acc.at[didx.at[gb]], add=True)
            return 0

        lax.fori_loop(0, NCH // 2, body, 0)
        last = NCH - 1
        pltpu.make_async_copy(h.at[sidx.at[last]], bufa, sem_a).wait()
        pltpu.sync_copy(bufa, acc.at[didx.at[last]], add=True)

    def do_wb(sout, p):
        for k in range(WPT // ZCH):
            start = WPT * sid + ZCH * k
            pltpu.sync_copy(acc.at[pl.ds(start, ZCH)], wb)
            pltpu.sync_copy(wb, sout.at[pl.ds(NHALF * p + start, ZCH)])

    for p, dstw in ((0, dst0w), (1, dst1w)):
        pltpu.sync_copy(dstw.at[sid], didx)
        _zero_wide(wb, ZCH, HH)
        for k in range(ZPT // ZCH):
            pltpu.sync_copy(wb, acc.at[pl.ds(ZPT * sid + ZCH * k, ZCH)])
        plsc.subcore_barrier()

        @pl.when(cid == 0)
        def _():
            do_edges(h0)

        @pl.when(cid == 1)
        def _():
            do_edges(h1)

        plsc.subcore_barrier()

        @pl.when(cid == 0)
        def _():
            do_wb(s0, p)

        @pl.when(cid == 1)
        def _():
            do_wb(s1, p)

        plsc.subcore_barrier()


def _b0_body(x_ref, w_ref, d0_ref, d1_ref, h0_ref, h1_ref, dv_ref):
    dinv = lax.rsqrt(0.5 * (d0_ref[...] + d1_ref[...]) + 1.0)
    dv_ref[...] = dinv
    hw = jnp.dot(x_ref[...], w_ref[...], preferred_element_type=jnp.float32)
    h0_ref[...] = hw[:, :HH] * dinv
    h1_ref[...] = hw[:, HH:] * dinv


def _b12_body(s0_ref, s1_ref, h0_ref, h1_ref, dv_ref, b_ref, w_ref,
              o0_ref, o1_ref):
    dinv = dv_ref[...]
    a0 = (s0_ref[...] + h0_ref[...]) * dinv + b_ref[:, :HH]
    a1 = (s1_ref[...] + h1_ref[...]) * dinv + b_ref[:, HH:]
    act = jnp.maximum(jnp.concatenate([a0, a1], axis=1), 0.0)
    hw = jnp.dot(act, w_ref[...], preferred_element_type=jnp.float32)
    o0_ref[...] = hw[:, :HH] * dinv
    o1_ref[...] = hw[:, HH:] * dinv


def _c_body(s0_ref, s1_ref, h0_ref, h1_ref, dv_ref, b_ref, batch_ref, r_ref):
    i = pl.program_id(0)
    dinv = dv_ref[...]
    a0 = (s0_ref[...] + h0_ref[...]) * dinv + b_ref[:, :HH]
    a1 = (s1_ref[...] + h1_ref[...]) * dinv + b_ref[:, HH:]
    act = jnp.maximum(jnp.concatenate([a0, a1], axis=1), 0.0)
    rows = i * BLK + lax.broadcasted_iota(jnp.int32, (BLK, 1), 0)
    act = jnp.where(rows < N, act, 0.0)
    oh = (batch_ref[...] == lax.broadcasted_iota(jnp.int32, (BLK, G), 1))
    contrib = lax.dot_general(oh.astype(jnp.float32), act,
                              (((0,), (0,)), ((), ())),
                              preferred_element_type=jnp.float32)

    @pl.when(i == 0)
    def _():
        r_ref[...] = contrib

    @pl.when(i > 0)
    def _():
        r_ref[...] += contrib


def _d_body(r_ref, w1_ref, b1_ref, w2_ref, b2_ref, w3_ref, b3_ref, out_ref):
    x = jnp.dot(r_ref[...], w1_ref[...], preferred_element_type=jnp.float32)
    x = jnp.maximum(x + b1_ref[...], 0.0)
    x = jnp.dot(x, w2_ref[...], preferred_element_type=jnp.float32)
    x = jnp.maximum(x + b2_ref[...], 0.0)
    out_ref[...] = (
        jnp.dot(x, w3_ref[...], preferred_element_type=jnp.float32)
        + b3_ref[...])


_spec_blk = lambda: pl.BlockSpec((BLK, HH), lambda i: (i, 0))
_spec_full = lambda shape: pl.BlockSpec(shape, lambda i: tuple(0 for _ in shape))

_b0_call = pl.pallas_call(
    _b0_body,
    grid=(NBLK,),
    in_specs=[
        pl.BlockSpec((BLK, D), lambda i: (i, 0)),
        _spec_full((D, D)),
        _spec_blk(), _spec_blk(),
    ],
    out_specs=[_spec_blk(), _spec_blk(), _spec_blk()],
    out_shape=[
        jax.ShapeDtypeStruct((NP, HH), jnp.float32),
        jax.ShapeDtypeStruct((NP, HH), jnp.float32),
        jax.ShapeDtypeStruct((NP, HH), jnp.float32),
    ],
)

_b12_call = pl.pallas_call(
    _b12_body,
    grid=(NBLK,),
    in_specs=[
        _spec_blk(), _spec_blk(), _spec_blk(), _spec_blk(), _spec_blk(),
        _spec_full((1, D)),
        _spec_full((D, D)),
    ],
    out_specs=[_spec_blk(), _spec_blk()],
    out_shape=[
        jax.ShapeDtypeStruct((NP, HH), jnp.float32),
        jax.ShapeDtypeStruct((NP, HH), jnp.float32),
    ],
)

_c_call = pl.pallas_call(
    _c_body,
    grid=(NBLK,),
    in_specs=[
        _spec_blk(), _spec_blk(), _spec_blk(), _spec_blk(), _spec_blk(),
        _spec_full((1, D)),
        pl.BlockSpec((BLK, G), lambda i: (i, 0)),
    ],
    out_specs=pl.BlockSpec((G, D), lambda i: (0, 0)),
    out_shape=jax.ShapeDtypeStruct((G, D), jnp.float32),
)

_d_call = pl.pallas_call(
    _d_body,
    out_shape=jax.ShapeDtypeStruct((G, 1), jnp.float32),
)


@jax.jit
def kernel(X, edge_index, batch, Wg0, bg0, Wg1, bg1, Wg2, bg2,
           Wc1, bc1, Wc2, bc2, Wc3, bc3):
    src = edge_index[0]
    dst = edge_index[1]
    src_s = src.reshape(TILES, NCH, CK)
    dst0f = jnp.where(dst < NHALF, dst, TRASH)
    dst1f = jnp.where(dst >= NHALF, dst - NHALF, TRASH)
    dst0 = dst0f.reshape(TILES, NCH, CK)
    dst1 = dst1f.reshape(TILES, NCH, CK)

    Xp = jnp.pad(X, ((0, NP - N), (0, 0)))
    batch_b = jnp.broadcast_to(
        jnp.pad(batch, (0, NP - N), constant_values=G)[:, None], (NP, G))

    ones_t = jnp.ones((8, HH), jnp.float32)
    zsrc = jnp.zeros((TILES, NCH, CK), jnp.int32)
    d0, d1 = _prop_sc(ones_t, ones_t, zsrc, dst0, dst1)

    h0, h1, dv = _b0_call(Xp, Wg0, d0, d1)
    s0, s1 = _prop_sc(h0, h1, src_s, dst0, dst1)
    h0, h1 = _b12_call(s0, s1, h0, h1, dv, bg0.reshape(1, D), Wg1)
    s0, s1 = _prop_sc(h0, h1, src_s, dst0, dst1)
    h0, h1 = _b12_call(s0, s1, h0, h1, dv, bg1.reshape(1, D), Wg2)
    s0, s1 = _prop_sc(h0, h1, src_s, dst0, dst1)
    r = _c_call(s0, s1, h0, h1, dv, bg2.reshape(1, D), batch_b)
    logits = _d_call(r, Wc1, bc1.reshape(1, D), Wc2, bc2.reshape(1, D),
                     Wc3, bc3.reshape(1, 1))
    return logits


# speedup vs baseline: 21.6859x; 21.6859x over previous
import functools

import jax
import jax.numpy as jnp
from jax import lax
from jax.experimental import pallas as pl
from jax.experimental.pallas import tpu as pltpu
from jax.experimental.pallas import tpu_sc as plsc

N = 10000
NP = 10240
E = 160000
D = 256
G = 128
HH = 128
BLK = 512
NBLK = NP // BLK

TILES = 16
CK = 80
EPT = E // TILES
NCH = EPT // CK
NHALF = NP // 2
ACCR = 6144
TRASH = 6000
ZPT = ACCR // TILES
WPT = NHALF // TILES
ZCH = 64

_MESH = plsc.VectorSubcoreMesh(core_axis_name="c", subcore_axis_name="s")


def _zero_wide(ref, rows, cols):
    def body(i, _):
        for j in range(cols // 16):
            ref[i, pl.ds(j * 16, 16)] = jnp.zeros((16,), jnp.float32)
        return 0
    lax.fori_loop(0, rows, body, 0)


@functools.partial(
    pl.kernel,
    mesh=_MESH,
    out_type=(
        jax.ShapeDtypeStruct((NP, HH), jnp.float32),
        jax.ShapeDtypeStruct((NP, HH), jnp.float32),
    ),
    scratch_types=[
        pltpu.VMEM((NCH, CK), jnp.int32),
        pltpu.VMEM((NCH, CK), jnp.int32),
        pltpu.VMEM((CK, HH), jnp.float32),
        pltpu.VMEM((CK, HH), jnp.float32),
        pltpu.VMEM((ZCH, HH), jnp.float32),
        pltpu.VMEM_SHARED((ACCR, HH), jnp.float32),
        pltpu.SemaphoreType.DMA,
        pltpu.SemaphoreType.DMA,
    ],
)
def _prop_sc(h0, h1, srcw, dst0w, dst1w, s0, s1, sidx, didx, bufa, bufb, wb,
             acc, sem_a, sem_b):
    cid = lax.axis_index("c")
    sid = lax.axis_index("s")

    pltpu.sync_copy(srcw.at[sid], sidx)

    def do_edges(h):
        pltpu.async_copy(h.at[sidx.at[0]], bufa, sem_a)

        def body(c, _):
            ga = 2 * c
            gb = 2 * c + 1
            pltpu.make_async_copy(h.at[sidx.at[ga]], bufa, sem_a).wait()
            pltpu.async_copy(h.at[sidx.at[gb]], bufb, sem_b)
            pltpu.sync_copy(bufa, acc.at[didx.at[ga]], add=True)
            pltpu.async_copy(h.at[sidx.at[ga + 2]], bufa, sem_a)
            pltpu.make_async_copy(h.at[sidx.at[gb]], bufb, sem_b).wait()
            pltpu.sync_copy(bufb, acc.at[didx.at[gb]], add=True)
            return 0

        lax.fori_loop(0, NCH // 2, body, 0)
        last = NCH - 1
        pltpu.make_async_copy(h.at[sidx.at[last]], bufa, sem_a).wait()
        pltpu.sync_copy(bufa, acc.at[didx.at[last]], add=True)

    def do_wb(sout, p):
        for k in range(WPT // ZCH):
            start = WPT * sid + ZCH * k
            pltpu.sync_copy(acc.at[pl.ds(start, ZCH)], wb)
            pltpu.sync_copy(wb, sout.at[pl.ds(NHALF * p + start, ZCH)])

    for p, dstw in ((0, dst0w), (1, dst1w)):
        pltpu.sync_copy(dstw.at[sid], didx)
        _zero_wide(wb, ZCH, HH)
        for k in range(ZPT // ZCH):
            pltpu.sync_copy(wb, acc.at[pl.ds(ZPT * sid + ZCH * k, ZCH)])
        plsc.subcore_barrier()

        @pl.when(cid == 0)
        def _():
            do_edges(h0)

        @pl.when(cid == 1)
        def _():
            do_edges(h1)

        plsc.subcore_barrier()

        @pl.when(cid == 0)
        def _():
            do_wb(s0, p)

        @pl.when(cid == 1)
        def _():
            do_wb(s1, p)

        plsc.subcore_barrier()


def _b0_body(x_ref, w_ref, d0_ref, d1_ref, h0_ref, h1_ref, dv_ref):
    dinv = lax.rsqrt(0.5 * (d0_ref[...] + d1_ref[...]) + 1.0)
    dv_ref[...] = dinv
    hw = jnp.dot(x_ref[...], w_ref[...], preferred_element_type=jnp.float32)
    h0_ref[...] = hw[:, :HH] * dinv
    h1_ref[...] = hw[:, HH:] * dinv


def _b12_body(s0_ref, s1_ref, h0_ref, h1_ref, dv_ref, b_ref, w_ref,
              o0_ref, o1_ref):
    dinv = dv_ref[...]
    a0 = (s0_ref[...] + h0_ref[...]) * dinv + b_ref[:, :HH]
    a1 = (s1_ref[...] + h1_ref[...]) * dinv + b_ref[:, HH:]
    act = jnp.maximum(jnp.concatenate([a0, a1], axis=1), 0.0)
    hw = jnp.dot(act, w_ref[...], preferred_element_type=jnp.float32)
    o0_ref[...] = hw[:, :HH] * dinv
    o1_ref[...] = hw[:, HH:] * dinv


def _c_body(s0_ref, s1_ref, h0_ref, h1_ref, dv_ref, b_ref, batch_ref, r_ref):
    i = pl.program_id(0)
    dinv = dv_ref[...]
    a0 = (s0_ref[...] + h0_ref[...]) * dinv + b_ref[:, :HH]
    a1 = (s1_ref[...] + h1_ref[...]) * dinv + b_ref[:, HH:]
    act = jnp.maximum(jnp.concatenate([a0, a1], axis=1), 0.0)
    rows = i * BLK + lax.broadcasted_iota(jnp.int32, (BLK, 1), 0)
    act = jnp.where(rows < N, act, 0.0)
    oh = (batch_ref[...] == lax.broadcasted_iota(jnp.int32, (BLK, G), 1))
    contrib = lax.dot_general(oh.astype(jnp.float32), act,
                              (((0,), (0,)), ((), ())),
                              preferred_element_type=jnp.float32)

    @pl.when(i == 0)
    def _():
        r_ref[...] = contrib

    @pl.when(i > 0)
    def _():
        r_ref[...] += contrib


def _d_body(r_ref, w1_ref, b1_ref, w2_ref, b2_ref, w3_ref, b3_ref, out_ref):
    x = jnp.dot(r_ref[...], w1_ref[...], preferred_element_type=jnp.float32)
    x = jnp.maximum(x + b1_ref[...], 0.0)
    x = jnp.dot(x, w2_ref[...], preferred_element_type=jnp.float32)
    x = jnp.maximum(x + b2_ref[...], 0.0)
    out_ref[...] = (
        jnp.dot(x, w3_ref[...], preferred_element_type=jnp.float32)
        + b3_ref[...])


_spec_blk = lambda: pl.BlockSpec((BLK, HH), lambda i: (i, 0))
_spec_full = lambda shape: pl.BlockSpec(shape, lambda i: tuple(0 for _ in shape))

_b0_call = pl.pallas_call(
    _b0_body,
    grid=(NBLK,),
    in_specs=[
        pl.BlockSpec((BLK, D), lambda i: (i, 0)),
        _spec_full((D, D)),
        _spec_blk(), _spec_blk(),
    ],
    out_specs=[_spec_blk(), _spec_blk(), _spec_blk()],
    out_shape=[
        jax.ShapeDtypeStruct((NP, HH), jnp.float32),
        jax.ShapeDtypeStruct((NP, HH), jnp.float32),
        jax.ShapeDtypeStruct((NP, HH), jnp.float32),
    ],
)

_b12_call = pl.pallas_call(
    _b12_body,
    grid=(NBLK,),
    in_specs=[
        _spec_blk(), _spec_blk(), _spec_blk(), _spec_blk(), _spec_blk(),
        _spec_full((1, D)),
        _spec_full((D, D)),
    ],
    out_specs=[_spec_blk(), _spec_blk()],
    out_shape=[
        jax.ShapeDtypeStruct((NP, HH), jnp.float32),
        jax.ShapeDtypeStruct((NP, HH), jnp.float32),
    ],
)

_c_call = pl.pallas_call(
    _c_body,
    grid=(NBLK,),
    in_specs=[
        _spec_blk(), _spec_blk(), _spec_blk(), _spec_blk(), _spec_blk(),
        _spec_full((1, D)),
        pl.BlockSpec((BLK, G), lambda i: (i, 0)),
    ],
    out_specs=pl.BlockSpec((G, D), lambda i: (0, 0)),
    out_shape=jax.ShapeDtypeStruct((G, D), jnp.float32),
)

_d_call = pl.pallas_call(
    _d_body,
    out_shape=jax.ShapeDtypeStruct((G, 1), jnp.float32),
)


@jax.jit
def kernel(X, edge_index, batch, Wg0, bg0, Wg1, bg1, Wg2, bg2,
           Wc1, bc1, Wc2, bc2, Wc3, bc3):
    src = edge_index[0]
    dst = edge_index[1]
    src_s = src.reshape(TILES, NCH, CK)
    dst0f = jnp.where(dst < NHALF, dst, TRASH)
    dst1f = jnp.where(dst >= NHALF, dst - NHALF, TRASH)
    dst0 = dst0f.reshape(TILES, NCH, CK)
    dst1 = dst1f.reshape(TILES, NCH, CK)

    Xp = jnp.pad(X, ((0, NP - N), (0, 0)))
    batch_b = jnp.broadcast_to(
        jnp.pad(batch, (0, NP - N), constant_values=G)[:, None], (NP, G))

    ones_t = jnp.ones((NP, HH), jnp.float32)
    d0, d1 = _prop_sc(ones_t, ones_t, src_s, dst0, dst1)

    h0, h1, dv = _b0_call(Xp, Wg0, d0, d1)
    s0, s1 = _prop_sc(h0, h1, src_s, dst0, dst1)
    h0, h1 = _b12_call(s0, s1, h0, h1, dv, bg0.reshape(1, D), Wg1)
    s0, s1 = _prop_sc(h0, h1, src_s, dst0, dst1)
    h0, h1 = _b12_call(s0, s1, h0, h1, dv, bg1.reshape(1, D), Wg2)
    s0, s1 = _prop_sc(h0, h1, src_s, dst0, dst1)
    r = _c_call(s0, s1, h0, h1, dv, bg2.reshape(1, D), batch_b)
    logits = _d_call(r, Wc1, bc1.reshape(1, D), Wc2, bc2.reshape(1, D),
                     Wc3, bc3.reshape(1, 1))
    return logits
